# baseline (speedup 1.0000x reference)
import jax
import jax.numpy as jnp
from jax import lax
from jax.experimental import pallas as pl

N_TOKENS = 1024
DIM = 256
N_Q = 4
BINS = 512
NSPLIT = 8
CHUNK = N_TOKENS // NSPLIT


def _split3_bf16(x):
    parts = []
    r = x
    for _ in range(3):
        c = r.astype(jnp.bfloat16)
        parts.append(c)
        r = r - c.astype(jnp.float32)
    return parts


def _dots(residual, cb):
    return lax.dot_general(
        residual, cb,
        dimension_numbers=(((1,), (1,)), ((), ())),
        preferred_element_type=jnp.float32,
        precision=lax.Precision.HIGHEST,
    )


def _select(dots, chalf_row):
    scores = dots - chalf_row
    maxs = jnp.max(scores, axis=1, keepdims=True)
    iota = lax.broadcasted_iota(jnp.int32, scores.shape, 1)
    idx2d = jnp.min(jnp.where(scores == maxs, iota, BINS),
                    axis=1, keepdims=True)
    onehot = (iota == idx2d).astype(jnp.bfloat16)
    return idx2d, onehot


def _gather(onehot, splits, lo):
    chosen = None
    for part in splits:
        g = lax.dot_general(
            onehot, part[lo:lo + BINS],
            dimension_numbers=(((1,), (0,)), ((), ())),
            preferred_element_type=jnp.float32,
        )
        chosen = g if chosen is None else chosen + g
    return chosen


def _rvq_kernel(h_ref, cb_ref, codes_ref, quant_ref):
    ones8 = jnp.ones((8, DIM), jnp.float32)
    cb_all = cb_ref[:].reshape(N_Q * BINS, DIM)
    chalf8 = 0.5 * lax.dot_general(
        ones8, cb_all * cb_all,
        dimension_numbers=(((1,), (1,)), ((), ())),
        preferred_element_type=jnp.float32,
        precision=lax.Precision.HIGHEST,
    )
    splits = _split3_bf16(cb_all)

    res = [h_ref[k * CHUNK:(k + 1) * CHUNK, :] for k in range(NSPLIT)]
    idx_cols = [[] for _ in range(NSPLIT)]
    for i in range(N_Q):
        cb = cb_ref[i]
        chalf_row = chalf8[0:1, i * BINS:(i + 1) * BINS]
        d = [_dots(res[k], cb) for k in range(NSPLIT)]
        for k in range(NSPLIT):
            idx_k, oh_k = _select(d[k], chalf_row)
            ch_k = _gather(oh_k, splits, i * BINS)
            res[k] = res[k] - ch_k
            idx_cols[k].append(idx_k)
    for k in range(NSPLIT):
        lo = k * CHUNK
        codes_ref[lo:lo + CHUNK, :] = jnp.concatenate(idx_cols[k], axis=1)
        quant_ref[lo:lo + CHUNK, :] = h_ref[lo:lo + CHUNK, :] - res[k]


def kernel(hidden_states, codebooks):
    codes_t, quant = pl.pallas_call(
        _rvq_kernel,
        out_shape=[
            jax.ShapeDtypeStruct((N_TOKENS, N_Q), jnp.int32),
            jax.ShapeDtypeStruct((N_TOKENS, DIM), jnp.float32),
        ],
    )(hidden_states, codebooks)
    return jnp.transpose(codes_t), quant


# speedup vs baseline: 1.3521x; 1.3521x over previous
import jax
import jax.numpy as jnp
from jax import lax
from jax.experimental import pallas as pl

N_TOKENS = 1024
DIM = 256
N_Q = 4
BINS = 512
NSPLIT = 4
CHUNK = N_TOKENS // NSPLIT


def _split3_bf16(x):
    parts = []
    r = x
    for _ in range(3):
        c = r.astype(jnp.bfloat16)
        parts.append(c)
        r = r - c.astype(jnp.float32)
    return parts


def _dots(residual, cb):
    return lax.dot_general(
        residual, cb,
        dimension_numbers=(((1,), (1,)), ((), ())),
        preferred_element_type=jnp.float32,
        precision=lax.Precision.HIGHEST,
    )


def _select(dots, chalf_row):
    scores = dots - chalf_row
    maxs = jnp.max(scores, axis=1, keepdims=True)
    iota = lax.broadcasted_iota(jnp.int32, scores.shape, 1)
    idx2d = jnp.min(jnp.where(scores == maxs, iota, BINS),
                    axis=1, keepdims=True)
    onehot = (iota == idx2d).astype(jnp.bfloat16)
    return idx2d, onehot


def _gather(onehot, splits, lo):
    chosen = None
    for part in splits:
        g = lax.dot_general(
            onehot, part[lo:lo + BINS],
            dimension_numbers=(((1,), (0,)), ((), ())),
            preferred_element_type=jnp.float32,
        )
        chosen = g if chosen is None else chosen + g
    return chosen


def _rvq_kernel(h_ref, cb_ref, codes_ref, quant_ref):
    ones8 = jnp.ones((8, DIM), jnp.float32)
    cb_all = cb_ref[:].reshape(N_Q * BINS, DIM)
    chalf8 = 0.5 * lax.dot_general(
        ones8, cb_all * cb_all,
        dimension_numbers=(((1,), (1,)), ((), ())),
        preferred_element_type=jnp.float32,
        precision=lax.Precision.HIGHEST,
    )
    splits = _split3_bf16(cb_all)

    res = [h_ref[k * CHUNK:(k + 1) * CHUNK, :] for k in range(NSPLIT)]
    idx_cols = [[] for _ in range(NSPLIT)]
    d = [_dots(res[k], cb_ref[0]) for k in range(NSPLIT)]
    for i in range(N_Q):
        chalf_row = chalf8[0:1, i * BINS:(i + 1) * BINS]
        for k in range(NSPLIT):
            idx_k, oh_k = _select(d[k], chalf_row)
            ch_k = _gather(oh_k, splits, i * BINS)
            res[k] = res[k] - ch_k
            if i + 1 < N_Q:
                d[k] = _dots(res[k], cb_ref[i + 1])
            idx_cols[k].append(idx_k)
    for k in range(NSPLIT):
        lo = k * CHUNK
        codes_ref[lo:lo + CHUNK, :] = jnp.concatenate(idx_cols[k], axis=1)
        quant_ref[lo:lo + CHUNK, :] = h_ref[lo:lo + CHUNK, :] - res[k]


def kernel(hidden_states, codebooks):
    codes_t, quant = pl.pallas_call(
        _rvq_kernel,
        out_shape=[
            jax.ShapeDtypeStruct((N_TOKENS, N_Q), jnp.int32),
            jax.ShapeDtypeStruct((N_TOKENS, DIM), jnp.float32),
        ],
    )(hidden_states, codebooks)
    return jnp.transpose(codes_t), quant


# speedup vs baseline: 1.3615x; 1.0070x over previous
import jax
import jax.numpy as jnp
from jax import lax
from jax.experimental import pallas as pl
from jax.experimental.pallas import tpu as pltpu

N_TOKENS = 1024
DIM = 256
N_Q = 4
BINS = 512
BLOCK_T = 512
NSPLIT = 2
CHUNK = BLOCK_T // NSPLIT


def _dots(residual, cb):
    return lax.dot_general(
        residual, cb,
        dimension_numbers=(((1,), (1,)), ((), ())),
        preferred_element_type=jnp.float32,
        precision=lax.Precision.HIGHEST,
    )


def _select(dots, chalf_row):
    scores = dots - chalf_row
    maxs = jnp.max(scores, axis=1, keepdims=True)
    iota = lax.broadcasted_iota(jnp.int32, scores.shape, 1)
    idx2d = jnp.min(jnp.where(scores == maxs, iota, BINS),
                    axis=1, keepdims=True)
    onehot = (iota == idx2d).astype(jnp.bfloat16)
    return idx2d, onehot


def _gather(onehot, splits_ref, lo):
    chosen = None
    for p in range(3):
        g = lax.dot_general(
            onehot, splits_ref[p, lo:lo + BINS],
            dimension_numbers=(((1,), (0,)), ((), ())),
            preferred_element_type=jnp.float32,
        )
        chosen = g if chosen is None else chosen + g
    return chosen


def _rvq_kernel(h_ref, cb_ref, codes_ref, quant_ref, splits_ref, chalf_ref):
    pid = pl.program_id(0)

    @pl.when(pid == 0)
    def _prep():
        cb_all = cb_ref[:].reshape(N_Q * BINS, DIM)
        ones8 = jnp.ones((8, DIM), jnp.float32)
        chalf_ref[:] = 0.5 * lax.dot_general(
            ones8, cb_all * cb_all,
            dimension_numbers=(((1,), (1,)), ((), ())),
            preferred_element_type=jnp.float32,
            precision=lax.Precision.HIGHEST,
        )
        r = cb_all
        for p in range(3):
            part = r.astype(jnp.bfloat16)
            splits_ref[p] = part
            r = r - part.astype(jnp.float32)

    res = [h_ref[k * CHUNK:(k + 1) * CHUNK, :] for k in range(NSPLIT)]
    idx_cols = [[] for _ in range(NSPLIT)]
    for i in range(N_Q):
        cb = cb_ref[i]
        chalf_row = chalf_ref[0:1, i * BINS:(i + 1) * BINS]
        d = [_dots(res[k], cb) for k in range(NSPLIT)]
        for k in range(NSPLIT):
            idx_k, oh_k = _select(d[k], chalf_row)
            ch_k = _gather(oh_k, splits_ref, i * BINS)
            res[k] = res[k] - ch_k
            idx_cols[k].append(idx_k)
    for k in range(NSPLIT):
        lo = k * CHUNK
        codes_ref[lo:lo + CHUNK, :] = jnp.concatenate(idx_cols[k], axis=1)
        quant_ref[lo:lo + CHUNK, :] = h_ref[lo:lo + CHUNK, :] - res[k]


def kernel(hidden_states, codebooks):
    codes_t, quant = pl.pallas_call(
        _rvq_kernel,
        grid=(N_TOKENS // BLOCK_T,),
        in_specs=[
            pl.BlockSpec((BLOCK_T, DIM), lambda j: (j, 0)),
            pl.BlockSpec((N_Q, BINS, DIM), lambda j: (0, 0, 0)),
        ],
        out_specs=[
            pl.BlockSpec((BLOCK_T, N_Q), lambda j: (j, 0)),
            pl.BlockSpec((BLOCK_T, DIM), lambda j: (j, 0)),
        ],
        out_shape=[
            jax.ShapeDtypeStruct((N_TOKENS, N_Q), jnp.int32),
            jax.ShapeDtypeStruct((N_TOKENS, DIM), jnp.float32),
        ],
        scratch_shapes=[
            pltpu.VMEM((3, N_Q * BINS, DIM), jnp.bfloat16),
            pltpu.VMEM((8, N_Q * BINS), jnp.float32),
        ],
    )(hidden_states, codebooks)
    return jnp.transpose(codes_t), quant


# speedup vs baseline: 1.6097x; 1.1823x over previous
import jax
import jax.numpy as jnp
from jax import lax
from jax.experimental import pallas as pl

N_TOKENS = 1024
DIM = 256
N_Q = 4
BINS = 512
NSPLIT = 4
CHUNK = N_TOKENS // NSPLIT


def _split3_bf16(x):
    parts = []
    r = x
    for _ in range(3):
        c = r.astype(jnp.bfloat16)
        parts.append(c)
        r = r - c.astype(jnp.float32)
    return parts


def _dots(residual, cb):
    return lax.dot_general(
        residual, cb,
        dimension_numbers=(((1,), (1,)), ((), ())),
        preferred_element_type=jnp.float32,
        precision=lax.Precision.HIGHEST,
    )


def _select(dots, chalf_row):
    scores = dots - chalf_row
    maxs = jnp.max(scores, axis=1, keepdims=True)
    iota = lax.broadcasted_iota(jnp.int32, scores.shape, 1)
    idx2d = jnp.min(jnp.where(scores == maxs, iota, BINS),
                    axis=1, keepdims=True)
    onehot = (iota == idx2d).astype(jnp.bfloat16)
    return idx2d, onehot


def _gather(onehot, splits, lo):
    chosen = None
    for part in splits:
        g = lax.dot_general(
            onehot, part[lo:lo + BINS],
            dimension_numbers=(((1,), (0,)), ((), ())),
            preferred_element_type=jnp.float32,
        )
        chosen = g if chosen is None else chosen + g
    return chosen


def _rvq_kernel(h_ref, cb_ref, codes_ref, quant_ref):
    ones8 = jnp.ones((8, DIM), jnp.float32)
    cb_all = cb_ref[:].reshape(N_Q * BINS, DIM)
    chalf8 = 0.5 * lax.dot_general(
        ones8, cb_all * cb_all,
        dimension_numbers=(((1,), (1,)), ((), ())),
        preferred_element_type=jnp.float32,
        precision=lax.Precision.HIGHEST,
    )
    splits = _split3_bf16(cb_all)

    res = [h_ref[k * CHUNK:(k + 1) * CHUNK, :] for k in range(NSPLIT)]
    idx_cols = [[] for _ in range(NSPLIT)]
    for i in range(N_Q):
        cb = cb_ref[i]
        chalf_row = chalf8[0:1, i * BINS:(i + 1) * BINS]
        d = [_dots(res[k], cb) for k in range(NSPLIT)]
        for k in range(NSPLIT):
            idx_k, oh_k = _select(d[k], chalf_row)
            ch_k = _gather(oh_k, splits, i * BINS)
            res[k] = res[k] - ch_k
            idx_cols[k].append(idx_k)
    for k in range(NSPLIT):
        lo = k * CHUNK
        codes_ref[lo:lo + CHUNK, :] = jnp.concatenate(idx_cols[k], axis=1)
        quant_ref[lo:lo + CHUNK, :] = h_ref[lo:lo + CHUNK, :] - res[k]


def kernel(hidden_states, codebooks):
    codes_t, quant = pl.pallas_call(
        _rvq_kernel,
        out_shape=[
            jax.ShapeDtypeStruct((N_TOKENS, N_Q), jnp.int32),
            jax.ShapeDtypeStruct((N_TOKENS, DIM), jnp.float32),
        ],
    )(hidden_states, codebooks)
    return jnp.transpose(codes_t), quant
